# baseline (speedup 1.0000x reference)
import functools

import jax
import jax.numpy as jnp
from jax.experimental import pallas as pl
from jax.experimental.pallas import tpu as pltpu

H = 1024
F = 2048
E = 8
K = 2
T = 2048

TB = 256
NTB = T // TB


def _router_body(x_ref, wgt_ref, cw_ref):
    logits = jax.lax.dot_general(
        x_ref[...].astype(jnp.bfloat16), wgt_ref[...].astype(jnp.bfloat16),
        (((1,), (0,)), ((), ())),
        preferred_element_type=jnp.float32)
    lane = jax.lax.broadcasted_iota(jnp.int32, (TB, E), 1)
    big = jnp.int32(E)
    l1 = jnp.max(logits, axis=1, keepdims=True)
    i1 = jnp.min(jnp.where(logits == l1, lane, big), axis=1, keepdims=True)
    masked = jnp.where(lane == i1, -jnp.inf, logits)
    l2 = jnp.max(masked, axis=1, keepdims=True)
    i2 = jnp.min(jnp.where(masked == l2, lane, big), axis=1, keepdims=True)
    w2 = 1.0 / (1.0 + jnp.exp(l1 - l2))
    w1 = 1.0 - w2
    cw_ref[...] = jnp.where(lane == i1, w1, 0.0) + jnp.where(lane == i2, w2, 0.0)


def _router(x, Wg):
    return pl.pallas_call(
        _router_body,
        grid=(NTB,),
        in_specs=[
            pl.BlockSpec((TB, H), lambda tb: (tb, 0)),
            pl.BlockSpec((H, E), lambda tb: (0, 0)),
        ],
        out_specs=pl.BlockSpec((TB, E), lambda tb: (tb, 0)),
        out_shape=jax.ShapeDtypeStruct((T, E), jnp.float32),
    )(x, Wg.T)


def _ffn_body(x_ref, w1_ref, w3_ref, w2_ref, cw_ref, out_ref):
    e = pl.program_id(0)
    tb = pl.program_id(1)
    xb = x_ref[...].astype(jnp.bfloat16)
    h = jnp.dot(xb, w1_ref[0], preferred_element_type=jnp.float32)
    g = jnp.dot(xb, w3_ref[0], preferred_element_type=jnp.float32)
    a = (h * jax.lax.logistic(h) * g).astype(jnp.bfloat16)
    y = jnp.dot(a, w2_ref[0], preferred_element_type=jnp.float32)
    lane = jax.lax.broadcasted_iota(jnp.int32, (TB, E), 1)
    cwe = jnp.sum(jnp.where(lane == e, cw_ref[...], 0.0), axis=1, keepdims=True)
    contrib = y * cwe
    sl = pl.ds(tb * TB, TB)

    @pl.when(e == 0)
    def _():
        out_ref[sl, :] = contrib

    @pl.when(e > 0)
    def _():
        out_ref[sl, :] += contrib


def _ffn_dense(x, cw, W1T, W3T, W2T):
    return pl.pallas_call(
        _ffn_body,
        grid=(E, NTB),
        in_specs=[
            pl.BlockSpec((TB, H), lambda e, tb: (tb, 0)),
            pl.BlockSpec((1, H, F), lambda e, tb: (e, 0, 0)),
            pl.BlockSpec((1, H, F), lambda e, tb: (e, 0, 0)),
            pl.BlockSpec((1, F, H), lambda e, tb: (e, 0, 0)),
            pl.BlockSpec((TB, E), lambda e, tb: (tb, 0)),
        ],
        out_specs=pl.BlockSpec((T, H), lambda e, tb: (0, 0)),
        out_shape=jax.ShapeDtypeStruct((T, H), jnp.float32),
        compiler_params=pltpu.CompilerParams(
            dimension_semantics=("arbitrary", "arbitrary")),
    )(x, W1T, W3T, W2T, cw)


def kernel(x, Wg, W1, W2, W3):
    W1T = jnp.transpose(W1, (0, 2, 1)).astype(jnp.bfloat16)
    W3T = jnp.transpose(W3, (0, 2, 1)).astype(jnp.bfloat16)
    W2T = jnp.transpose(W2, (0, 2, 1)).astype(jnp.bfloat16)
    cw = _router(x, Wg)
    return _ffn_dense(x, cw, W1T, W3T, W2T)
